# baseline (speedup 1.0000x reference)
import functools

import jax
import jax.numpy as jnp
from jax import lax
from jax.experimental import pallas as pl
from jax.experimental.pallas import tpu as pltpu
from jax.experimental.pallas import tpu_sc as plsc

N_POKES = 256
N_ABS = 212
N_ITEMS = 133
NHIDDEN = 128
B = 16384
L = 200

NTOK = B * L
CHUNK = 128
NROWS = NTOK // CHUNK
NC, NS = 2, 16
NW = NC * NS
ROWS_PER_W = NROWS // NW
K = 8
NBLK = ROWS_PER_W // K


def _make_sc_call():
    mesh = plsc.VectorSubcoreMesh(core_axis_name="c", subcore_axis_name="s")

    @functools.partial(
        pl.kernel,
        mesh=mesh,
        out_type=jax.ShapeDtypeStruct((NTOK, 3 * NHIDDEN), jnp.float32),
        scratch_types=[
            pltpu.VMEM((K, CHUNK), jnp.int32),
            pltpu.VMEM((K, CHUNK), jnp.int32),
            pltpu.VMEM((K, CHUNK), jnp.int32),
            pltpu.VMEM((2, CHUNK, NHIDDEN), jnp.float32),
            pltpu.VMEM((2, CHUNK, NHIDDEN), jnp.float32),
            pltpu.VMEM((2, CHUNK, NHIDDEN), jnp.float32),
            pltpu.SemaphoreType.DMA,
            pltpu.SemaphoreType.DMA,
            pltpu.SemaphoreType.DMA,
            pltpu.SemaphoreType.DMA,
            pltpu.VMEM_SHARED((N_POKES, NHIDDEN), jnp.float32),
            pltpu.VMEM_SHARED((N_ABS, NHIDDEN), jnp.float32),
            pltpu.VMEM_SHARED((N_ITEMS, NHIDDEN), jnp.float32),
        ],
    )
    def sc_kernel(poke_i, ab_i, item_i, pt, abt, itt, out,
                  idx_p, idx_a, idx_i, rows_p, rows_a, rows_i,
                  sem_g0, sem_g1, sem_w0, sem_w1, pt_sh, abt_sh, itt_sh):
        wid = lax.axis_index("s") * NC + lax.axis_index("c")
        row0 = wid * ROWS_PER_W
        sem_g = (sem_g0, sem_g1)
        sem_w = (sem_w0, sem_w1)

        @pl.when(lax.axis_index("s") == 0)
        def _stage():
            pltpu.sync_copy(pt, pt_sh)
            pltpu.sync_copy(abt, abt_sh)
            pltpu.sync_copy(itt, itt_sh)

        plsc.subcore_barrier()

        def blk(b, carry):
            r0 = row0 + b * K
            pltpu.sync_copy(poke_i.at[pl.ds(r0, K)], idx_p)
            pltpu.sync_copy(ab_i.at[pl.ds(r0, K)], idx_a)
            pltpu.sync_copy(item_i.at[pl.ds(r0, K)], idx_i)
            def fire_gathers(j, par):
                s = sem_g[par]
                return [
                    pltpu.async_copy(pt_sh.at[idx_p.at[j]], rows_p.at[par], s),
                    pltpu.async_copy(abt_sh.at[idx_a.at[j]], rows_a.at[par], s),
                    pltpu.async_copy(itt_sh.at[idx_i.at[j]], rows_i.at[par], s),
                ]

            def fire_writes(j, par):
                tok0 = (r0 + j) * CHUNK
                s = sem_w[par]
                return [
                    pltpu.async_copy(
                        rows_p.at[par],
                        out.at[pl.ds(tok0, CHUNK), pl.ds(0, NHIDDEN)], s),
                    pltpu.async_copy(
                        rows_a.at[par],
                        out.at[pl.ds(tok0, CHUNK), pl.ds(NHIDDEN, NHIDDEN)], s),
                    pltpu.async_copy(
                        rows_i.at[par],
                        out.at[pl.ds(tok0, CHUNK), pl.ds(2 * NHIDDEN, NHIDDEN)], s),
                ]

            pend_w = [[], []]
            pend_g = [[], []]
            pend_g[0] = fire_gathers(0, 0)
            for j in range(K):
                par = j % 2
                if j + 1 < K:
                    parn = (j + 1) % 2
                    for cw in pend_w[parn]:
                        cw.wait()
                    pend_w[parn] = []
                    pend_g[parn] = fire_gathers(j + 1, parn)
                for cg in pend_g[par]:
                    cg.wait()
                pend_g[par] = []
                pend_w[par] = fire_writes(j, par)
            for par in (0, 1):
                for cw in pend_w[par]:
                    cw.wait()
            return carry

        lax.fori_loop(0, NBLK, blk, 0)

    return sc_kernel


def kernel(poke_idx, ab_idx, item_idx, pokeEmb, abEmb, itemEmb):
    pi = poke_idx.astype(jnp.int32).reshape(NROWS, CHUNK)
    ai = ab_idx.astype(jnp.int32).reshape(NROWS, CHUNK)
    ii = item_idx.astype(jnp.int32).reshape(NROWS, CHUNK)
    out = _make_sc_call()(pi, ai, ii, pokeEmb, abEmb, itemEmb)
    return out.reshape(B, L, 3 * NHIDDEN)


# speedup vs baseline: 1.0490x; 1.0490x over previous
import functools

import jax
import jax.numpy as jnp
from jax import lax
from jax.experimental import pallas as pl
from jax.experimental.pallas import tpu as pltpu
from jax.experimental.pallas import tpu_sc as plsc

N_POKES = 256
N_ABS = 212
N_ITEMS = 133
NHIDDEN = 128
B = 16384
L = 200

NTOK = B * L
CHUNK = 128
NROWS = NTOK // CHUNK
NC, NS = 2, 16
NW = NC * NS
ROWS_PER_W = NROWS // NW
K = 8
NBLK = ROWS_PER_W // K


def _make_sc_call():
    mesh = plsc.VectorSubcoreMesh(core_axis_name="c", subcore_axis_name="s")

    @functools.partial(
        pl.kernel,
        mesh=mesh,
        out_type=jax.ShapeDtypeStruct((NTOK, 3 * NHIDDEN), jnp.float32),
        scratch_types=[
            pltpu.VMEM((K, CHUNK), jnp.int32),
            pltpu.VMEM((K, CHUNK), jnp.int32),
            pltpu.VMEM((K, CHUNK), jnp.int32),
            pltpu.VMEM((2, CHUNK, NHIDDEN), jnp.float32),
            pltpu.VMEM((2, CHUNK, NHIDDEN), jnp.float32),
            pltpu.VMEM((2, CHUNK, NHIDDEN), jnp.float32),
            pltpu.SemaphoreType.DMA,
            pltpu.SemaphoreType.DMA,
            pltpu.SemaphoreType.DMA,
            pltpu.VMEM_SHARED((N_POKES, NHIDDEN), jnp.float32),
            pltpu.VMEM_SHARED((N_ABS, NHIDDEN), jnp.float32),
            pltpu.VMEM_SHARED((N_ITEMS, NHIDDEN), jnp.float32),
        ],
    )
    def sc_kernel(poke_i, ab_i, item_i, pt, abt, itt, out,
                  idx_p, idx_a, idx_i, rows_p, rows_a, rows_i,
                  sem_g, sem_w0, sem_w1, pt_sh, abt_sh, itt_sh):
        wid = lax.axis_index("s") * NC + lax.axis_index("c")
        row0 = wid * ROWS_PER_W
        sem_w = (sem_w0, sem_w1)

        @pl.when(lax.axis_index("s") == 0)
        def _stage():
            pltpu.sync_copy(pt, pt_sh)
            pltpu.sync_copy(abt, abt_sh)
            pltpu.sync_copy(itt, itt_sh)

        plsc.subcore_barrier()

        def blk(b, carry):
            r0 = row0 + b * K
            pltpu.sync_copy(poke_i.at[pl.ds(r0, K)], idx_p)
            pltpu.sync_copy(ab_i.at[pl.ds(r0, K)], idx_a)
            pltpu.sync_copy(item_i.at[pl.ds(r0, K)], idx_i)
            pending = [[], []]
            for j in range(K):
                par = j % 2
                for cw in pending[par]:
                    cw.wait()
                pending[par] = []
                cp = pltpu.async_copy(pt_sh.at[idx_p.at[j]], rows_p.at[par], sem_g)
                ca = pltpu.async_copy(abt_sh.at[idx_a.at[j]], rows_a.at[par], sem_g)
                ci = pltpu.async_copy(itt_sh.at[idx_i.at[j]], rows_i.at[par], sem_g)
                cp.wait()
                ca.wait()
                ci.wait()
                tok0 = (r0 + j) * CHUNK
                s = sem_w[par]
                pending[par] = [
                    pltpu.async_copy(
                        rows_p.at[par],
                        out.at[pl.ds(tok0, CHUNK), pl.ds(0, NHIDDEN)], s),
                    pltpu.async_copy(
                        rows_a.at[par],
                        out.at[pl.ds(tok0, CHUNK), pl.ds(NHIDDEN, NHIDDEN)], s),
                    pltpu.async_copy(
                        rows_i.at[par],
                        out.at[pl.ds(tok0, CHUNK), pl.ds(2 * NHIDDEN, NHIDDEN)], s),
                ]
            for par in (0, 1):
                for cw in pending[par]:
                    cw.wait()
            return carry

        lax.fori_loop(0, NBLK, blk, 0)

    return sc_kernel


def kernel(poke_idx, ab_idx, item_idx, pokeEmb, abEmb, itemEmb):
    pi = poke_idx.astype(jnp.int32).reshape(NROWS, CHUNK)
    ai = ab_idx.astype(jnp.int32).reshape(NROWS, CHUNK)
    ii = item_idx.astype(jnp.int32).reshape(NROWS, CHUNK)
    out = _make_sc_call()(pi, ai, ii, pokeEmb, abEmb, itemEmb)
    return out.reshape(B, L, 3 * NHIDDEN)


# speedup vs baseline: 1.1191x; 1.0668x over previous
import functools

import jax
import jax.numpy as jnp
from jax import lax
from jax.experimental import pallas as pl
from jax.experimental.pallas import tpu as pltpu
from jax.experimental.pallas import tpu_sc as plsc

N_POKES = 256
N_ABS = 212
N_ITEMS = 133
NHIDDEN = 128
B = 16384
L = 200

NTOK = B * L
CHUNK = 128
NROWS = NTOK // CHUNK
NC, NS = 2, 16
NW = NC * NS
ROWS_PER_W = NROWS // NW
K = 16
NBLK = ROWS_PER_W // K


def _make_sc_call():
    mesh = plsc.VectorSubcoreMesh(core_axis_name="c", subcore_axis_name="s")

    @functools.partial(
        pl.kernel,
        mesh=mesh,
        out_type=jax.ShapeDtypeStruct((NTOK, 3 * NHIDDEN), jnp.float32),
        scratch_types=[
            pltpu.VMEM((K, CHUNK), jnp.int32),
            pltpu.VMEM((K, CHUNK), jnp.int32),
            pltpu.VMEM((K, CHUNK), jnp.int32),
            pltpu.VMEM((2, CHUNK, NHIDDEN), jnp.float32),
            pltpu.VMEM((2, CHUNK, NHIDDEN), jnp.float32),
            pltpu.VMEM((2, CHUNK, NHIDDEN), jnp.float32),
            pltpu.SemaphoreType.DMA,
            pltpu.SemaphoreType.DMA,
            pltpu.SemaphoreType.DMA,
            pltpu.VMEM_SHARED((N_POKES, NHIDDEN), jnp.float32),
            pltpu.VMEM_SHARED((N_ABS, NHIDDEN), jnp.float32),
            pltpu.VMEM_SHARED((N_ITEMS, NHIDDEN), jnp.float32),
        ],
    )
    def sc_kernel(poke_i, ab_i, item_i, pt, abt, itt, out,
                  idx_p, idx_a, idx_i, rows_p, rows_a, rows_i,
                  sem_g, sem_w0, sem_w1, pt_sh, abt_sh, itt_sh):
        wid = lax.axis_index("s") * NC + lax.axis_index("c")
        row0 = wid * ROWS_PER_W
        sem_w = (sem_w0, sem_w1)

        @pl.when(lax.axis_index("s") == 0)
        def _stage():
            pltpu.sync_copy(pt, pt_sh)
            pltpu.sync_copy(abt, abt_sh)
            pltpu.sync_copy(itt, itt_sh)

        plsc.subcore_barrier()

        def blk(b, carry):
            r0 = row0 + b * K
            pltpu.sync_copy(poke_i.at[pl.ds(r0, K)], idx_p)
            pltpu.sync_copy(ab_i.at[pl.ds(r0, K)], idx_a)
            pltpu.sync_copy(item_i.at[pl.ds(r0, K)], idx_i)
            pending = [[], []]
            for j in range(K):
                par = j % 2
                for cw in pending[par]:
                    cw.wait()
                pending[par] = []
                cp = pltpu.async_copy(pt_sh.at[idx_p.at[j]], rows_p.at[par], sem_g)
                ca = pltpu.async_copy(abt_sh.at[idx_a.at[j]], rows_a.at[par], sem_g)
                ci = pltpu.async_copy(itt_sh.at[idx_i.at[j]], rows_i.at[par], sem_g)
                cp.wait()
                ca.wait()
                ci.wait()
                tok0 = (r0 + j) * CHUNK
                s = sem_w[par]
                pending[par] = [
                    pltpu.async_copy(
                        rows_p.at[par],
                        out.at[pl.ds(tok0, CHUNK), pl.ds(0, NHIDDEN)], s),
                    pltpu.async_copy(
                        rows_a.at[par],
                        out.at[pl.ds(tok0, CHUNK), pl.ds(NHIDDEN, NHIDDEN)], s),
                    pltpu.async_copy(
                        rows_i.at[par],
                        out.at[pl.ds(tok0, CHUNK), pl.ds(2 * NHIDDEN, NHIDDEN)], s),
                ]
            for par in (0, 1):
                for cw in pending[par]:
                    cw.wait()
            return carry

        lax.fori_loop(0, NBLK, blk, 0)

    return sc_kernel


def kernel(poke_idx, ab_idx, item_idx, pokeEmb, abEmb, itemEmb):
    pi = poke_idx.astype(jnp.int32).reshape(NROWS, CHUNK)
    ai = ab_idx.astype(jnp.int32).reshape(NROWS, CHUNK)
    ii = item_idx.astype(jnp.int32).reshape(NROWS, CHUNK)
    out = _make_sc_call()(pi, ai, ii, pokeEmb, abEmb, itemEmb)
    return out.reshape(B, L, 3 * NHIDDEN)
